# baseline (speedup 1.0000x reference)
import jax
import jax.numpy as jnp
from jax.experimental import pallas as pl
from jax.experimental.pallas import tpu as pltpu

_N_GRAPHS = 512
_K = 8
_SUB = 8
_STEP = _K * _SUB


def _shaper_block(*refs):
    ns, ms, coefs_ref, out_ref = refs[:_K], refs[_K:2 * _K], refs[-2], refs[-1]
    for k, (nk, mk) in enumerate(zip(ns, ms)):
        x = nk[...]
        cm = mk[...]
        c = coefs_ref[pl.ds(k * _SUB, _SUB), :]
        c0 = c[:, 0:1]
        c1 = c[:, 1:2]
        c2 = c[:, 2:3]
        c3 = c[:, 3:4]
        t = c0 + x * (c1 + x * (c2 + x * c3))
        d = t - cm
        sq = d * d
        out_ref[pl.ds(k * _SUB, _SUB), :] = (
            jnp.sum(sq, axis=1, keepdims=True) * (1.0 / _N_GRAPHS))


def kernel(neuron_mat, concept_mat, coefs, graph_idxs):
    del graph_idxs
    n_concepts, n_nodes = neuron_mat.shape
    nr = n_concepts // _STEP
    assert nr * _STEP == n_concepts

    def sub_spec(k):
        return pl.BlockSpec((_SUB, n_nodes), lambda i, k=k: (_K * i + k, 0))

    out = pl.pallas_call(
        _shaper_block,
        grid=(nr,),
        in_specs=(
            [sub_spec(k) for k in range(_K)]
            + [sub_spec(k) for k in range(_K)]
            + [pl.BlockSpec((_STEP, coefs.shape[1]), lambda i: (i, 0))]
        ),
        out_specs=pl.BlockSpec((_STEP, 1), lambda i: (i, 0)),
        out_shape=jax.ShapeDtypeStruct((n_concepts, 1), jnp.float32),
        compiler_params=pltpu.CompilerParams(
            dimension_semantics=("parallel",)),
    )(*([neuron_mat] * _K + [concept_mat] * _K + [coefs]))
    return out[:, 0]


# speedup vs baseline: 1.0252x; 1.0252x over previous
import jax
import jax.numpy as jnp
from jax.experimental import pallas as pl
from jax.experimental.pallas import tpu as pltpu

_N_GRAPHS = 512
_K = 4
_SUB = 8
_STEP = _K * _SUB


def _shaper_block(*refs):
    (n0, n1, n2, n3, m0, m1, m2, m3, coefs_ref, out_ref) = refs
    for k, (nk, mk) in enumerate(((n0, m0), (n1, m1), (n2, m2), (n3, m3))):
        x = nk[...]
        cm = mk[...]
        c = coefs_ref[pl.ds(k * _SUB, _SUB), :]
        c0 = c[:, 0:1]
        c1 = c[:, 1:2]
        c2 = c[:, 2:3]
        c3 = c[:, 3:4]
        t = c0 + x * (c1 + x * (c2 + x * c3))
        d = t - cm
        sq = d * d
        out_ref[pl.ds(k * _SUB, _SUB), :] = (
            jnp.sum(sq, axis=1, keepdims=True) * (1.0 / _N_GRAPHS))


def kernel(neuron_mat, concept_mat, coefs, graph_idxs):
    del graph_idxs
    n_concepts, n_nodes = neuron_mat.shape
    nr = n_concepts // _STEP
    assert nr * _STEP == n_concepts

    def sub_spec(k):
        return pl.BlockSpec((_SUB, n_nodes), lambda i, k=k: (_K * i + k, 0))

    out = pl.pallas_call(
        _shaper_block,
        grid=(nr,),
        in_specs=(
            [sub_spec(k) for k in range(_K)]
            + [sub_spec(k) for k in range(_K)]
            + [pl.BlockSpec((_STEP, coefs.shape[1]), lambda i: (i, 0))]
        ),
        out_specs=pl.BlockSpec((_STEP, 1), lambda i: (i, 0)),
        out_shape=jax.ShapeDtypeStruct((n_concepts, 1), jnp.float32),
        compiler_params=pltpu.CompilerParams(
            dimension_semantics=("parallel",)),
    )(*([neuron_mat] * _K + [concept_mat] * _K + [coefs]))
    return out[:, 0]
